# Initial kernel scaffold; baseline (speedup 1.0000x reference)
import jax
import jax.numpy as jnp
from jax.experimental import pallas as pl


def kernel(x0, edge_index0, edge_attr0, batch0, x1, edge_index1, edge_attr1, batch1, atom_emb1, atom_emb2, edge_emb1, edge_emb2, W1, b1, W2, b2, bn_scale, bn_bias, Wp1, bp1, Wp2, bp2, Wc1, bc1, Wc2, bc2):
    raise NotImplementedError("write your pallas kernel here")


# speedup vs baseline: 4.8043x; 4.8043x over previous
import functools

import jax
import jax.numpy as jnp
from jax import lax
from jax.experimental import pallas as pl
from jax.experimental.pallas import tpu as pltpu
from jax.experimental.pallas import tpu_sc as plsc

N = 10000
E = 160000
EMB = 300
NLAYER = 5
NGRAPH = 64

NP = 10240
HALF = 160
IN = 2 * HALF
HID = 640
BLK = 256
NBLK = NP // BLK
EBLK = 640
NTILE = 16
NRT = NP // NTILE

K = 80
EPT = E // NTILE
NCH = EPT // K

K2 = 40
EPC = E // (2 * NTILE)
NCH2 = EPC // K2

_f32 = jnp.float32


def _spmm_body(hl, hr, src, dst, sl_out, sr_out, src_v, dst_v, rows_v, acc,
               sem):
    cid = lax.axis_index("c")
    sid = lax.axis_index("s")
    rbase = sid * NRT

    @pl.when(cid == 0)
    def _():
        pltpu.sync_copy(hl.at[pl.ds(rbase, NRT)], acc.at[pl.ds(rbase, NRT)])

    @pl.when(cid == 1)
    def _():
        pltpu.sync_copy(hr.at[pl.ds(rbase, NRT)], acc.at[pl.ds(rbase, NRT)])

    plsc.subcore_barrier()

    ebase = sid * EPT

    def body(ch, carry):
        eb = ebase + ch * K
        pltpu.sync_copy(src.at[pl.ds(eb, K)], src_v)
        pltpu.sync_copy(dst.at[pl.ds(eb, K)], dst_v)

        @pl.when(cid == 0)
        def _():
            pltpu.async_copy(hl.at[src_v], rows_v, sem).wait()

        @pl.when(cid == 1)
        def _():
            pltpu.async_copy(hr.at[src_v], rows_v, sem).wait()

        pltpu.sync_copy(rows_v, acc.at[dst_v], add=True)
        return carry

    lax.fori_loop(0, NCH, body, 0)
    plsc.subcore_barrier()

    @pl.when(cid == 0)
    def _():
        pltpu.sync_copy(acc.at[pl.ds(rbase, NRT)], sl_out.at[pl.ds(rbase, NRT)])

    @pl.when(cid == 1)
    def _():
        pltpu.sync_copy(acc.at[pl.ds(rbase, NRT)], sr_out.at[pl.ds(rbase, NRT)])


def _counts_body(oh, dst, zeros, c0_out, c1_out, dst_v, rows_v, acc):
    cid = lax.axis_index("c")
    sid = lax.axis_index("s")
    rbase = sid * NRT
    pltpu.sync_copy(zeros.at[pl.ds(rbase, NRT)], acc.at[pl.ds(rbase, NRT)])
    plsc.subcore_barrier()

    ebase = cid * (E // 2) + sid * EPC

    def body(ch, carry):
        eb = ebase + ch * K2
        pltpu.sync_copy(dst.at[pl.ds(eb, K2)], dst_v)
        pltpu.sync_copy(oh.at[pl.ds(eb, K2)], rows_v)
        pltpu.sync_copy(rows_v, acc.at[dst_v], add=True)
        return carry

    lax.fori_loop(0, NCH2, body, 0)
    plsc.subcore_barrier()

    @pl.when(cid == 0)
    def _():
        pltpu.sync_copy(acc.at[pl.ds(rbase, NRT)], c0_out.at[pl.ds(rbase, NRT)])

    @pl.when(cid == 1)
    def _():
        pltpu.sync_copy(acc.at[pl.ds(rbase, NRT)], c1_out.at[pl.ds(rbase, NRT)])


@functools.lru_cache(maxsize=None)
def _sc_kernels():
    mesh = plsc.VectorSubcoreMesh(core_axis_name="c", subcore_axis_name="s",
                                  num_cores=2, num_subcores=NTILE)
    params = pltpu.CompilerParams(use_tc_tiling_on_sc=False)
    spmm = pl.kernel(
        _spmm_body,
        mesh=mesh,
        compiler_params=params,
        out_type=[jax.ShapeDtypeStruct((NP, HALF), _f32),
                  jax.ShapeDtypeStruct((NP, HALF), _f32)],
        scratch_types=[
            pltpu.VMEM((K,), jnp.int32),
            pltpu.VMEM((K,), jnp.int32),
            pltpu.VMEM((K, HALF), _f32),
            pltpu.VMEM_SHARED((NP, HALF), _f32),
            pltpu.SemaphoreType.DMA,
        ],
    )
    counts = pl.kernel(
        _counts_body,
        mesh=mesh,
        compiler_params=params,
        out_type=[jax.ShapeDtypeStruct((NP, 16), _f32),
                  jax.ShapeDtypeStruct((NP, 16), _f32)],
        scratch_types=[
            pltpu.VMEM((K2,), jnp.int32),
            pltpu.VMEM((K2, 16), _f32),
            pltpu.VMEM_SHARED((NP, 16), _f32),
        ],
    )
    return spmm, counts


def _emb_body(x_ref, t9l, t9r, ol, orr):
    xv = x_ref[...]
    cls = xv[:, 0:1] * 3 + xv[:, 1:2]
    oh = (cls == lax.broadcasted_iota(jnp.int32, (BLK, 16), 1)).astype(_f32)
    ol[...] = jnp.dot(oh, t9l[...], preferred_element_type=_f32, precision=lax.Precision.HIGHEST)
    orr[...] = jnp.dot(oh, t9r[...], preferred_element_type=_f32, precision=lax.Precision.HIGHEST)


_emb = pl.pallas_call(
    _emb_body,
    grid=(NBLK,),
    in_specs=[
        pl.BlockSpec((BLK, 2), lambda i: (i, 0)),
        pl.BlockSpec((16, HALF), lambda i: (0, 0)),
        pl.BlockSpec((16, HALF), lambda i: (0, 0)),
    ],
    out_specs=[
        pl.BlockSpec((BLK, HALF), lambda i: (i, 0)),
        pl.BlockSpec((BLK, HALF), lambda i: (i, 0)),
    ],
    out_shape=[jax.ShapeDtypeStruct((NP, HALF), _f32),
               jax.ShapeDtypeStruct((NP, HALF), _f32)],
)


def _ohe_body(ea_ref, oh_ref):
    ev = ea_ref[...]
    cls = ev[:, 0:1] * 3 + ev[:, 1:2]
    oh_ref[...] = (cls == lax.broadcasted_iota(jnp.int32, (EBLK, 16), 1)).astype(_f32)


_ohe = pl.pallas_call(
    _ohe_body,
    grid=(E // EBLK,),
    in_specs=[pl.BlockSpec((EBLK, 2), lambda i: (i, 0))],
    out_specs=pl.BlockSpec((EBLK, 16), lambda i: (i, 0)),
    out_shape=jax.ShapeDtypeStruct((E, 16), _f32),
)


def _pass1_body(sl, sr, c0, c1, tel, ter, w1l, w1r, b1, w2l, w2r, b2l, b2r,
                hhl_ref, hhr_ref, s1l_ref, s1r_ref, s2l_ref, s2r_ref):
    i = pl.program_id(0)
    cb = c0[...] + c1[...]
    aggl = sl[...] + jnp.dot(cb, tel[...], preferred_element_type=_f32, precision=lax.Precision.HIGHEST)
    aggr = sr[...] + jnp.dot(cb, ter[...], preferred_element_type=_f32, precision=lax.Precision.HIGHEST)
    u = (jnp.dot(aggl, w1l[...], preferred_element_type=_f32, precision=lax.Precision.HIGHEST)
         + jnp.dot(aggr, w1r[...], preferred_element_type=_f32, precision=lax.Precision.HIGHEST) + b1[...])
    u = jnp.maximum(u, 0.0)
    rid = i * BLK + lax.broadcasted_iota(jnp.int32, (BLK, 1), 0)
    rmask = rid < N

    @pl.when(i == 0)
    def _():
        s1l_ref[...] = jnp.zeros_like(s1l_ref)
        s1r_ref[...] = jnp.zeros_like(s1r_ref)
        s2l_ref[...] = jnp.zeros_like(s2l_ref)
        s2r_ref[...] = jnp.zeros_like(s2r_ref)

    def half(w2, b2, hh_ref, s1_ref, s2_ref):
        hh = jnp.dot(u, w2[...], preferred_element_type=_f32, precision=lax.Precision.HIGHEST) + b2[...]
        hh_ref[...] = hh
        hm = jnp.where(rmask, hh, 0.0)
        s1_ref[...] += jnp.broadcast_to(
            jnp.sum(hm, axis=0, keepdims=True), (8, HALF))
        s2_ref[...] += jnp.broadcast_to(
            jnp.sum(hm * hm, axis=0, keepdims=True), (8, HALF))

    half(w2l, b2l, hhl_ref, s1l_ref, s2l_ref)
    half(w2r, b2r, hhr_ref, s1r_ref, s2r_ref)


_pass1 = pl.pallas_call(
    _pass1_body,
    grid=(NBLK,),
    in_specs=[
        pl.BlockSpec((BLK, HALF), lambda i: (i, 0)),
        pl.BlockSpec((BLK, HALF), lambda i: (i, 0)),
        pl.BlockSpec((BLK, 16), lambda i: (i, 0)),
        pl.BlockSpec((BLK, 16), lambda i: (i, 0)),
        pl.BlockSpec((16, HALF), lambda i: (0, 0)),
        pl.BlockSpec((16, HALF), lambda i: (0, 0)),
        pl.BlockSpec((HALF, HID), lambda i: (0, 0)),
        pl.BlockSpec((HALF, HID), lambda i: (0, 0)),
        pl.BlockSpec((1, HID), lambda i: (0, 0)),
        pl.BlockSpec((HID, HALF), lambda i: (0, 0)),
        pl.BlockSpec((HID, HALF), lambda i: (0, 0)),
        pl.BlockSpec((1, HALF), lambda i: (0, 0)),
        pl.BlockSpec((1, HALF), lambda i: (0, 0)),
    ],
    out_specs=[
        pl.BlockSpec((BLK, HALF), lambda i: (i, 0)),
        pl.BlockSpec((BLK, HALF), lambda i: (i, 0)),
        pl.BlockSpec((8, HALF), lambda i: (0, 0)),
        pl.BlockSpec((8, HALF), lambda i: (0, 0)),
        pl.BlockSpec((8, HALF), lambda i: (0, 0)),
        pl.BlockSpec((8, HALF), lambda i: (0, 0)),
    ],
    out_shape=[jax.ShapeDtypeStruct((NP, HALF), _f32),
               jax.ShapeDtypeStruct((NP, HALF), _f32),
               jax.ShapeDtypeStruct((8, HALF), _f32),
               jax.ShapeDtypeStruct((8, HALF), _f32),
               jax.ShapeDtypeStruct((8, HALF), _f32),
               jax.ShapeDtypeStruct((8, HALF), _f32)],
)


def _pass2_body(relu, hhl, hhr, s1l, s1r, s2l, s2r, scl, scr, bil, bir,
                ol, orr):
    def norm(hh, s1, s2, sc, bi):
        m = s1[0:1, :] * (1.0 / N)
        var = s2[0:1, :] * (1.0 / N) - m * m
        inv = lax.rsqrt(var + 1e-5)
        y = (hh[...] - m) * inv * sc[...] + bi[...]
        return jnp.maximum(y, 0.0) if relu else y

    ol[...] = norm(hhl, s1l, s2l, scl, bil)
    orr[...] = norm(hhr, s1r, s2r, scr, bir)


def _make_pass2(relu):
    return pl.pallas_call(
        functools.partial(_pass2_body, relu),
        grid=(NBLK,),
        in_specs=[
            pl.BlockSpec((BLK, HALF), lambda i: (i, 0)),
            pl.BlockSpec((BLK, HALF), lambda i: (i, 0)),
            pl.BlockSpec((8, HALF), lambda i: (0, 0)),
            pl.BlockSpec((8, HALF), lambda i: (0, 0)),
            pl.BlockSpec((8, HALF), lambda i: (0, 0)),
            pl.BlockSpec((8, HALF), lambda i: (0, 0)),
            pl.BlockSpec((1, HALF), lambda i: (0, 0)),
            pl.BlockSpec((1, HALF), lambda i: (0, 0)),
            pl.BlockSpec((1, HALF), lambda i: (0, 0)),
            pl.BlockSpec((1, HALF), lambda i: (0, 0)),
        ],
        out_specs=[
            pl.BlockSpec((BLK, HALF), lambda i: (i, 0)),
            pl.BlockSpec((BLK, HALF), lambda i: (i, 0)),
        ],
        out_shape=[jax.ShapeDtypeStruct((NP, HALF), _f32),
                   jax.ShapeDtypeStruct((NP, HALF), _f32)],
    )


_pass2_relu = _make_pass2(True)
_pass2_final = _make_pass2(False)


def _pp_body(hl, hr, bt, w1l, w1r, b1, w2, b2, pool_ref):
    i = pl.program_id(0)
    u = jnp.maximum(
        jnp.dot(hl[...], w1l[...], preferred_element_type=_f32, precision=lax.Precision.HIGHEST)
        + jnp.dot(hr[...], w1r[...], preferred_element_type=_f32, precision=lax.Precision.HIGHEST) + b1[...],
        0.0)
    o = jnp.dot(u, w2[...], preferred_element_type=_f32, precision=lax.Precision.HIGHEST) + b2[...]
    oh = (bt[...] == lax.broadcasted_iota(jnp.int32, (BLK, NGRAPH), 1)
          ).astype(_f32)

    @pl.when(i == 0)
    def _():
        pool_ref[...] = jnp.zeros_like(pool_ref)

    pool_ref[...] += lax.dot_general(
        oh, o, (((0,), (0,)), ((), ())), preferred_element_type=_f32, precision=lax.Precision.HIGHEST)


_projpool = pl.pallas_call(
    _pp_body,
    grid=(NBLK,),
    in_specs=[
        pl.BlockSpec((BLK, HALF), lambda i: (i, 0)),
        pl.BlockSpec((BLK, HALF), lambda i: (i, 0)),
        pl.BlockSpec((BLK, 1), lambda i: (i, 0)),
        pl.BlockSpec((HALF, IN), lambda i: (0, 0)),
        pl.BlockSpec((HALF, IN), lambda i: (0, 0)),
        pl.BlockSpec((1, IN), lambda i: (0, 0)),
        pl.BlockSpec((IN, IN), lambda i: (0, 0)),
        pl.BlockSpec((1, IN), lambda i: (0, 0)),
    ],
    out_specs=pl.BlockSpec((NGRAPH, IN), lambda i: (0, 0)),
    out_shape=jax.ShapeDtypeStruct((NGRAPH, IN), _f32),
)


def _clf_body(p0, p1, wc1, bc1, wc2, bc2, out_ref):
    a0 = p0[...]
    a1 = p1[...]
    f0 = a0 / jnp.maximum(a0[:, EMB:EMB + 1], 1.0)
    f1 = a1 / jnp.maximum(a1[:, EMB:EMB + 1], 1.0)
    f2 = pltpu.roll(f1, 1, 0)

    def head(g):
        u = jnp.maximum(
            jnp.dot(g, wc1[...], preferred_element_type=_f32, precision=lax.Precision.HIGHEST) + bc1[...], 0.0)
        return jnp.dot(u, wc2[...], preferred_element_type=_f32, precision=lax.Precision.HIGHEST) + bc2[...]

    out_ref[...] = jnp.concatenate(
        [head(jnp.maximum(f0, f1)), head(jnp.maximum(f0, f2))], axis=0)


_clf = pl.pallas_call(
    _clf_body,
    grid=(1,),
    in_specs=[
        pl.BlockSpec((NGRAPH, IN), lambda i: (0, 0)),
        pl.BlockSpec((NGRAPH, IN), lambda i: (0, 0)),
        pl.BlockSpec((IN, IN), lambda i: (0, 0)),
        pl.BlockSpec((1, IN), lambda i: (0, 0)),
        pl.BlockSpec((IN, 128), lambda i: (0, 0)),
        pl.BlockSpec((1, 128), lambda i: (0, 0)),
    ],
    out_specs=pl.BlockSpec((2 * NGRAPH, 128), lambda i: (0, 0)),
    out_shape=jax.ShapeDtypeStruct((2 * NGRAPH, 128), _f32),
)


def kernel(x0, edge_index0, edge_attr0, batch0,
           x1, edge_index1, edge_attr1, batch1,
           atom_emb1, atom_emb2, edge_emb1, edge_emb2,
           W1, b1, W2, b2, bn_scale, bn_bias,
           Wp1, bp1, Wp2, bp2, Wc1, bc1, Wc2, bc2):
    a9 = jnp.arange(9) // 3
    b9 = jnp.arange(9) % 3
    t9 = atom_emb1[a9] + atom_emb2[b9]
    t9p = jnp.zeros((16, IN), _f32).at[:9, :EMB].set(t9)
    te = edge_emb1[:, a9] + edge_emb2[:, b9]
    tep = jnp.zeros((NLAYER, 16, IN), _f32).at[:, :9, :EMB].set(te)
    esl = edge_emb1[:, 4] + edge_emb2[:, 0]

    w1p = jnp.zeros((NLAYER, IN, HID), _f32).at[:, :EMB, :2 * EMB].set(W1)
    b1e = b1 + jnp.einsum("le,leh->lh", esl, W1)
    b1p = jnp.zeros((NLAYER, 1, HID), _f32).at[:, 0, :2 * EMB].set(b1e)
    w2p = jnp.zeros((NLAYER, HID, IN), _f32).at[:, :2 * EMB, :EMB].set(W2)
    b2p = jnp.zeros((NLAYER, 1, IN), _f32).at[:, 0, :EMB].set(b2)
    scp = jnp.zeros((NLAYER, 1, IN), _f32).at[:, 0, :EMB].set(bn_scale)
    bip = jnp.zeros((NLAYER, 1, IN), _f32).at[:, 0, :EMB].set(bn_bias)

    wp1p = jnp.zeros((IN, IN), _f32).at[:EMB, :EMB].set(Wp1)
    bp1p = jnp.zeros((1, IN), _f32).at[0, :EMB].set(bp1)
    wp2p = jnp.zeros((IN, IN), _f32).at[:EMB, :EMB].set(Wp2)
    bp2p = jnp.zeros((1, IN), _f32).at[0, :EMB].set(bp2).at[0, EMB].set(1.0)

    wc1p = jnp.zeros((IN, IN), _f32).at[:EMB, :EMB].set(Wc1)
    bc1p = jnp.zeros((1, IN), _f32).at[0, :EMB].set(bc1)
    wc2p = jnp.zeros((IN, 128), _f32).at[:EMB, 0].set(Wc2[:, 0])
    bc2p = jnp.zeros((1, 128), _f32).at[0, 0].set(bc2[0])

    zeros16 = jnp.zeros((NP, 16), _f32)
    spmm, counts = _sc_kernels()

    def run_graph(x, ei, ea, batch):
        xpad = jnp.pad(x, ((0, NP - N), (0, 0)))
        bt = jnp.pad(batch, (0, NP - N),
                     constant_values=NGRAPH).reshape(NP, 1)
        src = ei[0]
        dst = ei[1]
        hl, hr = _emb(xpad, t9p[:, :HALF], t9p[:, HALF:])
        ohm = _ohe(ea)
        c0, c1 = counts(ohm, dst, zeros16)
        for l in range(NLAYER):
            sl, sr = spmm(hl, hr, src, dst)
            hhl, hhr, s1l, s1r, s2l, s2r = _pass1(
                sl, sr, c0, c1,
                tep[l, :, :HALF], tep[l, :, HALF:],
                w1p[l, :HALF], w1p[l, HALF:], b1p[l],
                w2p[l, :, :HALF], w2p[l, :, HALF:],
                b2p[l, :, :HALF], b2p[l, :, HALF:])
            p2 = _pass2_relu if l < NLAYER - 1 else _pass2_final
            hl, hr = p2(hhl, hhr, s1l, s1r, s2l, s2r,
                        scp[l, :, :HALF], scp[l, :, HALF:],
                        bip[l, :, :HALF], bip[l, :, HALF:])
        return _projpool(hl, hr, bt, wp1p[:HALF], wp1p[HALF:],
                         bp1p, wp2p, bp2p)

    p0 = run_graph(x0, edge_index0, edge_attr0, batch0)
    p1 = run_graph(x1, edge_index1, edge_attr1, batch1)
    out = _clf(p0, p1, wc1p, bc1p, wc2p, bc2p)
    logits = out[:, 0]
    labels = jnp.concatenate([jnp.ones((NGRAPH,), _f32),
                              jnp.zeros((NGRAPH,), _f32)], axis=0)
    return logits, labels
